# Initial kernel scaffold; baseline (speedup 1.0000x reference)
import jax
import jax.numpy as jnp
from jax.experimental import pallas as pl


def kernel(x, low, high):
    raise NotImplementedError("write your pallas kernel here")


# speedup vs baseline: 31.9367x; 31.9367x over previous
import functools

import jax
import jax.numpy as jnp
from jax import lax
from jax.experimental import pallas as pl
from jax.experimental.pallas import tpu as pltpu
from jax.experimental.pallas import tpu_sc as plsc

N = 4096 * 4096
NC, NS = 2, 16
NW = NC * NS
PER_W = N // NW
CHUNK = 16384
CHUNKS = PER_W // CHUNK
VPC = CHUNK // 16
B1 = 65536
B2 = 32768

DECAY = 0.99
I_LO, FRAC_LO = 838860, 0.75
I_HI, FRAC_HI = 15938354, 0.25


def _keyify(u, mint, c31):
    m = lax.shift_right_arithmetic(u, c31)
    return lax.bitwise_xor(u, lax.bitwise_or(m, mint))


def _sc_pass1(x_hbm, out_hbm, buf, hist):
    wid = lax.axis_index("s") * NC + lax.axis_index("c")
    zero16 = jnp.zeros((16,), jnp.int32)
    ones16 = jnp.ones((16,), jnp.int32)
    mint = jnp.full((16,), -(2**31), jnp.int32)
    c31 = jnp.full((16,), 31, jnp.int32)
    c16 = jnp.full((16,), 16, jnp.int32)

    @pl.loop(0, B1 // 16)
    def _zero(i):
        hist[pl.ds(i * 16, 16)] = zero16

    base = wid * PER_W

    @pl.loop(0, CHUNKS)
    def _chunk(ci):
        pltpu.sync_copy(x_hbm.at[pl.ds(base + ci * CHUNK, CHUNK)], buf)

        @pl.loop(0, VPC)
        def _vec(vi):
            u = buf[pl.ds(vi * 16, 16)]
            key = _keyify(u, mint, c31)
            b = lax.shift_right_logical(key, c16)
            plsc.addupdate_scatter(hist, [b], ones16)

    pltpu.sync_copy(hist, out_hbm.at[wid])


def _sc_pass2(x_hbm, meta_hbm, out_hbm, buf, hist_a, hist_b, mv):
    wid = lax.axis_index("s") * NC + lax.axis_index("c")
    zero16 = jnp.zeros((16,), jnp.int32)
    ones16 = jnp.ones((16,), jnp.int32)
    mint = jnp.full((16,), -(2**31), jnp.int32)
    c31 = jnp.full((16,), 31, jnp.int32)
    c16 = jnp.full((16,), 16, jnp.int32)
    c2 = jnp.full((16,), 2, jnp.int32)
    m14 = jnp.full((16,), 0x3FFF, jnp.int32)
    sel = jnp.full((16,), 16384, jnp.int32)

    pltpu.sync_copy(meta_hbm, mv)
    p0lo = mv[pl.ds(0, 16)]
    p1lo = mv[pl.ds(16, 16)]
    p0hi = mv[pl.ds(32, 16)]
    p1hi = mv[pl.ds(48, 16)]

    @pl.loop(0, B2 // 16)
    def _zero(i):
        hist_a[pl.ds(i * 16, 16)] = zero16
        hist_b[pl.ds(i * 16, 16)] = zero16

    base = wid * PER_W

    @pl.loop(0, CHUNKS)
    def _chunk(ci):
        pltpu.sync_copy(x_hbm.at[pl.ds(base + ci * CHUNK, CHUNK)], buf)

        @pl.loop(0, VPC)
        def _vec(vi):
            u = buf[pl.ds(vi * 16, 16)]
            key = _keyify(u, mint, c31)
            pfx = lax.shift_right_logical(key, c16)
            low14 = lax.bitwise_and(lax.shift_right_logical(key, c2), m14)
            eq0 = pfx == p0lo
            eq1 = pfx == p1lo
            bn = jnp.where(eq1, lax.bitwise_or(low14, sel), low14)
            plsc.addupdate_scatter(hist_a, [bn], ones16,
                                   mask=lax.bitwise_or(eq0, eq1))
            eq0h = pfx == p0hi
            eq1h = pfx == p1hi
            bnh = jnp.where(eq1h, lax.bitwise_or(low14, sel), low14)
            plsc.addupdate_scatter(hist_b, [bnh], ones16,
                                   mask=lax.bitwise_or(eq0h, eq1h))

    pltpu.sync_copy(hist_a, out_hbm.at[wid, 0])
    pltpu.sync_copy(hist_b, out_hbm.at[wid, 1])


def _cumsum_2d(h):
    rows, cols = h.shape
    cs = h
    k = 1
    while k < cols:
        z = jnp.zeros((rows, k), h.dtype)
        cs = cs + jnp.concatenate([z, cs[:, :cols - k]], axis=1)
        k *= 2
    rt = cs[:, cols - 1:cols]
    rc = rt
    k = 1
    while k < rows:
        z = jnp.zeros((k, 1), h.dtype)
        rc = rc + jnp.concatenate([z, rc[:rows - k, :]], axis=0)
        k *= 2
    return cs + rc - rt


def _tc_prefix(h_ref, o_ref):
    hs = jnp.sum(h_ref[...], axis=0)
    cum = _cumsum_2d(hs)
    binid = (lax.broadcasted_iota(jnp.int32, (512, 128), 0) * 128
             + lax.broadcasted_iota(jnp.int32, (512, 128), 1))

    def find(i):
        p = jnp.sum((cum <= i).astype(jnp.int32))
        below = jnp.sum(jnp.where(binid == p, cum - hs, 0))
        return p, below

    p0lo, cb_lo = find(I_LO)
    p1lo, _ = find(I_LO + 1)
    p0hi, cb_hi = find(I_HI)
    p1hi, _ = find(I_HI + 1)
    vals = [p0lo, p1lo, p0hi, p1hi, I_LO - cb_lo, I_HI - cb_hi]
    lane = lax.broadcasted_iota(jnp.int32, (1, 128), 1)
    o = jnp.zeros((1, 128), jnp.int32)
    for k, v in enumerate(vals):
        o = jnp.where(lane == k, v, o)
    o_ref[...] = o


def _tc_final(h_ref, m_ref, lh_ref, o_ref):
    m = m_ref[...]
    lane = lax.broadcasted_iota(jnp.int32, (1, 128), 1)

    def ilane(k):
        return jnp.sum(jnp.where(lane == k, m, 0))

    lh = lh_ref[...]
    low = jnp.sum(jnp.where(lane == 0, lh, 0.0))
    high = jnp.sum(jnp.where(lane == 1, lh, 0.0))
    hsum = jnp.sum(h_ref[...], axis=0)

    def quantile(t, p0, p1, r, frac):
        cum = _cumsum_2d(hsum[t])

        def pick(rank):
            b = jnp.sum((cum <= rank).astype(jnp.int32))
            in_hi = lax.shift_right_logical(b, 14)
            low14 = lax.bitwise_and(b, 0x3FFF)
            pfx = jnp.where(in_hi == 1, p1, p0)
            key = lax.bitwise_or(lax.shift_left(pfx, 16),
                                 lax.shift_left(low14, 2))
            bits = jnp.where(key < 0, lax.bitwise_and(key, 0x7FFFFFFF),
                             lax.bitwise_not(key))
            return lax.bitcast_convert_type(bits, jnp.float32)

        fa = pick(r)
        fb = pick(r + 1)
        return fa + jnp.float32(frac) * (fb - fa)

    q_lo = quantile(0, ilane(0), ilane(1), ilane(4), FRAC_LO)
    q_hi = quantile(1, ilane(2), ilane(3), ilane(5), FRAC_HI)
    new_low = jnp.float32(DECAY) * low + jnp.float32(1.0 - DECAY) * q_lo
    new_high = jnp.float32(DECAY) * high + jnp.float32(1.0 - DECAY) * q_hi
    inv = jnp.maximum(jnp.float32(1.0), new_high - new_low)
    o = jnp.where(lane == 0, new_low, jnp.where(lane == 1, inv, 0.0))
    o_ref[...] = o.astype(jnp.float32)


@functools.cache
def _sc_kernels():
    mesh = plsc.VectorSubcoreMesh(core_axis_name="c", subcore_axis_name="s",
                                  num_cores=NC, num_subcores=NS)
    params = pltpu.CompilerParams(needs_layout_passes=False)
    pass1 = pl.kernel(
        _sc_pass1,
        out_type=jax.ShapeDtypeStruct((NW, B1), jnp.int32),
        mesh=mesh,
        compiler_params=params,
        scratch_types=[
            pltpu.VMEM((CHUNK,), jnp.int32),
            pltpu.VMEM((B1,), jnp.int32),
        ],
    )
    pass2 = pl.kernel(
        _sc_pass2,
        out_type=jax.ShapeDtypeStruct((NW, 2, B2), jnp.int32),
        mesh=mesh,
        compiler_params=params,
        scratch_types=[
            pltpu.VMEM((CHUNK,), jnp.int32),
            pltpu.VMEM((B2,), jnp.int32),
            pltpu.VMEM((B2,), jnp.int32),
            pltpu.VMEM((64,), jnp.int32),
        ],
    )
    return pass1, pass2


_prefix = pl.pallas_call(
    _tc_prefix,
    out_shape=jax.ShapeDtypeStruct((1, 128), jnp.int32),
)

_final = pl.pallas_call(
    _tc_final,
    out_shape=jax.ShapeDtypeStruct((1, 128), jnp.float32),
)


def kernel(x, low, high):
    x_i32 = lax.bitcast_convert_type(x, jnp.int32).reshape(N)
    _pass1, _pass2 = _sc_kernels()
    hist1 = _pass1(x_i32)
    meta = _prefix(hist1.reshape(NW, 512, 128))
    meta64 = jnp.repeat(meta[0, :4], 16)
    hist2 = _pass2(x_i32, meta64)
    lh = jnp.zeros((1, 128), jnp.float32).at[0, 0].set(low).at[0, 1].set(high)
    out = _final(hist2.reshape(NW, 2, 256, 128), meta, lh)
    return (out[0, 0], out[0, 1])


# speedup vs baseline: 38.8109x; 1.2152x over previous
import functools

import jax
import jax.numpy as jnp
from jax import lax
from jax.experimental import pallas as pl
from jax.experimental.pallas import tpu as pltpu
from jax.experimental.pallas import tpu_sc as plsc

N = 4096 * 4096
COLS = 4096
NC, NS = 2, 16
NW = NC * NS
PER_W = N // NW
ROWS_W = 4096 // NW
ROWS_C = 4
CHUNK = ROWS_C * COLS
CHUNKS = ROWS_W // ROWS_C
VPR = COLS // 16
B1 = 65536
B2 = 32768

DECAY = 0.99
I_LO, FRAC_LO = 838860, 0.75
I_HI, FRAC_HI = 15938354, 0.25


def _keyify(u, mint, c31):
    m = lax.shift_right_arithmetic(u, c31)
    return lax.bitwise_xor(u, lax.bitwise_or(m, mint))


def _double_buffered_scan(x_hbm, base_row, buf0, buf1, sem0, sem1, process):
    def sl(ci):
        return pl.ds(base_row + ci * ROWS_C, ROWS_C)

    pltpu.async_copy(x_hbm.at[sl(0)], buf0, sem0)

    @pl.loop(0, CHUNKS, step=2)
    def _chunk(ci):
        pltpu.async_copy(x_hbm.at[sl(ci + 1)], buf1, sem1)
        pltpu.make_async_copy(x_hbm.at[sl(ci)], buf0, sem0).wait()
        process(buf0)

        @pl.when(ci + 2 < CHUNKS)
        def _():
            pltpu.async_copy(x_hbm.at[sl(ci + 2)], buf0, sem0)

        pltpu.make_async_copy(x_hbm.at[sl(ci + 1)], buf1, sem1).wait()
        process(buf1)


def _sc_pass1(x_hbm, out_hbm, buf0, buf1, hist, sem0, sem1):
    wid = lax.axis_index("s") * NC + lax.axis_index("c")
    zero16 = jnp.zeros((16,), jnp.int32)
    ones16 = jnp.ones((16,), jnp.int32)
    mint = jnp.full((16,), -(2**31), jnp.int32)
    c31 = jnp.full((16,), 31, jnp.int32)
    c16 = jnp.full((16,), 16, jnp.int32)

    @pl.loop(0, B1 // 16, unroll=8)
    def _zero(i):
        hist[pl.ds(i * 16, 16)] = zero16

    def process(buf):
        for r in range(ROWS_C):
            @pl.loop(0, VPR, unroll=8)
            def _vec(vi):
                u = buf[r, pl.ds(vi * 16, 16)]
                key = _keyify(u, mint, c31)
                b = lax.shift_right_logical(key, c16)
                plsc.addupdate_scatter(hist, [b], ones16)

    _double_buffered_scan(x_hbm, wid * ROWS_W, buf0, buf1, sem0, sem1, process)
    pltpu.sync_copy(hist, out_hbm.at[wid])


def _sc_pass2(x_hbm, meta_hbm, out_hbm, buf0, buf1, hist_a, hist_b, mv,
              sem0, sem1):
    wid = lax.axis_index("s") * NC + lax.axis_index("c")
    zero16 = jnp.zeros((16,), jnp.int32)
    ones16 = jnp.ones((16,), jnp.int32)
    mint = jnp.full((16,), -(2**31), jnp.int32)
    c31 = jnp.full((16,), 31, jnp.int32)
    c16 = jnp.full((16,), 16, jnp.int32)
    c2 = jnp.full((16,), 2, jnp.int32)
    m14 = jnp.full((16,), 0x3FFF, jnp.int32)
    sel = jnp.full((16,), 16384, jnp.int32)

    pltpu.sync_copy(meta_hbm, mv)
    p0lo = mv[pl.ds(0, 16)]
    p1lo = mv[pl.ds(16, 16)]
    p0hi = mv[pl.ds(32, 16)]
    p1hi = mv[pl.ds(48, 16)]

    @pl.loop(0, B2 // 16, unroll=8)
    def _zero(i):
        hist_a[pl.ds(i * 16, 16)] = zero16
        hist_b[pl.ds(i * 16, 16)] = zero16

    def process(buf):
        for r in range(ROWS_C):
            @pl.loop(0, VPR, unroll=8)
            def _vec(vi):
                u = buf[r, pl.ds(vi * 16, 16)]
                key = _keyify(u, mint, c31)
                pfx = lax.shift_right_logical(key, c16)
                low14 = lax.bitwise_and(lax.shift_right_logical(key, c2), m14)
                eq0 = pfx == p0lo
                eq1 = pfx == p1lo
                bn = jnp.where(eq1, lax.bitwise_or(low14, sel), low14)
                plsc.addupdate_scatter(hist_a, [bn], ones16,
                                       mask=lax.bitwise_or(eq0, eq1))
                eq0h = pfx == p0hi
                eq1h = pfx == p1hi
                bnh = jnp.where(eq1h, lax.bitwise_or(low14, sel), low14)
                plsc.addupdate_scatter(hist_b, [bnh], ones16,
                                       mask=lax.bitwise_or(eq0h, eq1h))

    _double_buffered_scan(x_hbm, wid * ROWS_W, buf0, buf1, sem0, sem1, process)
    pltpu.sync_copy(hist_a, out_hbm.at[wid, 0])
    pltpu.sync_copy(hist_b, out_hbm.at[wid, 1])


def _cumsum_2d(h):
    rows, cols = h.shape
    cs = h
    k = 1
    while k < cols:
        z = jnp.zeros((rows, k), h.dtype)
        cs = cs + jnp.concatenate([z, cs[:, :cols - k]], axis=1)
        k *= 2
    rt = cs[:, cols - 1:cols]
    rc = rt
    k = 1
    while k < rows:
        z = jnp.zeros((k, 1), h.dtype)
        rc = rc + jnp.concatenate([z, rc[:rows - k, :]], axis=0)
        k *= 2
    return cs + rc - rt


def _tc_prefix(h_ref, o_ref):
    hs = jnp.sum(h_ref[...], axis=0)
    cum = _cumsum_2d(hs)
    binid = (lax.broadcasted_iota(jnp.int32, (512, 128), 0) * 128
             + lax.broadcasted_iota(jnp.int32, (512, 128), 1))

    def find(i):
        p = jnp.sum((cum <= i).astype(jnp.int32))
        below = jnp.sum(jnp.where(binid == p, cum - hs, 0))
        return p, below

    p0lo, cb_lo = find(I_LO)
    p1lo, _ = find(I_LO + 1)
    p0hi, cb_hi = find(I_HI)
    p1hi, _ = find(I_HI + 1)
    vals = [p0lo, p1lo, p0hi, p1hi, I_LO - cb_lo, I_HI - cb_hi]
    lane = lax.broadcasted_iota(jnp.int32, (1, 128), 1)
    o = jnp.zeros((1, 128), jnp.int32)
    for k, v in enumerate(vals):
        o = jnp.where(lane == k, v, o)
    o_ref[...] = o


def _tc_final(h_ref, m_ref, lh_ref, o_ref):
    m = m_ref[...]
    lane = lax.broadcasted_iota(jnp.int32, (1, 128), 1)

    def ilane(k):
        return jnp.sum(jnp.where(lane == k, m, 0))

    lh = lh_ref[...]
    low = jnp.sum(jnp.where(lane == 0, lh, 0.0))
    high = jnp.sum(jnp.where(lane == 1, lh, 0.0))
    hsum = jnp.sum(h_ref[...], axis=0)

    def quantile(t, p0, p1, r, frac):
        cum = _cumsum_2d(hsum[t])

        def pick(rank):
            b = jnp.sum((cum <= rank).astype(jnp.int32))
            in_hi = lax.shift_right_logical(b, 14)
            low14 = lax.bitwise_and(b, 0x3FFF)
            pfx = jnp.where(in_hi == 1, p1, p0)
            key = lax.bitwise_or(lax.shift_left(pfx, 16),
                                 lax.shift_left(low14, 2))
            bits = jnp.where(key < 0, lax.bitwise_and(key, 0x7FFFFFFF),
                             lax.bitwise_not(key))
            return lax.bitcast_convert_type(bits, jnp.float32)

        fa = pick(r)
        fb = pick(r + 1)
        return fa + jnp.float32(frac) * (fb - fa)

    q_lo = quantile(0, ilane(0), ilane(1), ilane(4), FRAC_LO)
    q_hi = quantile(1, ilane(2), ilane(3), ilane(5), FRAC_HI)
    new_low = jnp.float32(DECAY) * low + jnp.float32(1.0 - DECAY) * q_lo
    new_high = jnp.float32(DECAY) * high + jnp.float32(1.0 - DECAY) * q_hi
    inv = jnp.maximum(jnp.float32(1.0), new_high - new_low)
    o = jnp.where(lane == 0, new_low, jnp.where(lane == 1, inv, 0.0))
    o_ref[...] = o.astype(jnp.float32)


@functools.cache
def _sc_kernels():
    mesh = plsc.VectorSubcoreMesh(core_axis_name="c", subcore_axis_name="s",
                                  num_cores=NC, num_subcores=NS)
    params = pltpu.CompilerParams(needs_layout_passes=False)
    pass1 = pl.kernel(
        _sc_pass1,
        out_type=jax.ShapeDtypeStruct((NW, B1), jnp.int32),
        name="sc_hist_pass1",
        mesh=mesh,
        compiler_params=params,
        scratch_types=[
            pltpu.VMEM((ROWS_C, COLS), jnp.int32),
            pltpu.VMEM((ROWS_C, COLS), jnp.int32),
            pltpu.VMEM((B1,), jnp.int32),
            pltpu.SemaphoreType.DMA,
            pltpu.SemaphoreType.DMA,
        ],
    )
    pass2 = pl.kernel(
        _sc_pass2,
        out_type=jax.ShapeDtypeStruct((NW, 2, B2), jnp.int32),
        name="sc_hist_pass2",
        mesh=mesh,
        compiler_params=params,
        scratch_types=[
            pltpu.VMEM((ROWS_C, COLS), jnp.int32),
            pltpu.VMEM((ROWS_C, COLS), jnp.int32),
            pltpu.VMEM((B2,), jnp.int32),
            pltpu.VMEM((B2,), jnp.int32),
            pltpu.VMEM((64,), jnp.int32),
            pltpu.SemaphoreType.DMA,
            pltpu.SemaphoreType.DMA,
        ],
    )
    return pass1, pass2


_prefix = pl.pallas_call(
    _tc_prefix,
    out_shape=jax.ShapeDtypeStruct((1, 128), jnp.int32),
)

_final = pl.pallas_call(
    _tc_final,
    out_shape=jax.ShapeDtypeStruct((1, 128), jnp.float32),
)


def kernel(x, low, high):
    x_i32 = lax.bitcast_convert_type(x, jnp.int32)
    _pass1, _pass2 = _sc_kernels()
    hist1 = _pass1(x_i32)
    meta = _prefix(hist1.reshape(NW, 512, 128))
    meta64 = jnp.repeat(meta[0, :4], 16)
    hist2 = _pass2(x_i32, meta64)
    lh = jnp.zeros((1, 128), jnp.float32).at[0, 0].set(low).at[0, 1].set(high)
    out = _final(hist2.reshape(NW, 2, 256, 128), meta, lh)
    return (out[0, 0], out[0, 1])
